# Initial kernel scaffold; baseline (speedup 1.0000x reference)
import jax
import jax.numpy as jnp
from jax.experimental import pallas as pl


def kernel(pair_i):
    raise NotImplementedError("write your pallas kernel here")


# speedup vs baseline: 1.5814x; 1.5814x over previous
import functools

import jax
import jax.numpy as jnp
from jax import lax
from jax.experimental import pallas as pl
from jax.experimental.pallas import tpu as pltpu
from jax.experimental.pallas import tpu_sc as plsc

N_ATOMS = 100000
N_PAIRS = 6400000
L = 16

PAD_N = 106496

CHUNK = 10000


def _sc_partial_hists(pair_i):
  mesh = plsc.VectorSubcoreMesh(core_axis_name="c", subcore_axis_name="s")
  nw = mesh.num_cores * mesh.num_subcores
  per_w = N_PAIRS // nw
  n_chunks = per_w // CHUNK

  @functools.partial(
      pl.kernel,
      out_type=jax.ShapeDtypeStruct((nw, PAD_N), jnp.int32),
      mesh=mesh,
      compiler_params=pltpu.CompilerParams(needs_layout_passes=False),
      scratch_types=[
          pltpu.VMEM((PAD_N,), jnp.int32),
          pltpu.VMEM((CHUNK,), jnp.int32),
          pltpu.VMEM((CHUNK,), jnp.int32),
          pltpu.SemaphoreType.DMA,
          pltpu.SemaphoreType.DMA,
      ],
  )
  def hist_kernel(x_hbm, out_hbm, hist, buf0, buf1, sem0, sem1):
    wid = lax.axis_index("s") * mesh.num_cores + lax.axis_index("c")
    base = wid * per_w

    bufs = (buf0, buf1)
    sems = (sem0, sem1)
    descs = [None] * n_chunks
    descs[0] = pltpu.async_copy(
        x_hbm.at[pl.ds(base, CHUNK)], buf0, sem0)

    zv = jnp.zeros((L,), jnp.int32)

    def zero_body(j, c):
      hist[pl.ds(j * L, L)] = zv
      return c

    lax.fori_loop(0, PAD_N // L, zero_body, 0, unroll=8)

    ones = jnp.ones((L,), jnp.int32)

    def count_body(buf):
      def body(i, c):
        v = buf[pl.ds(i * L, L)]
        plsc.addupdate_scatter(hist, [v], ones)
        return c
      lax.fori_loop(0, CHUNK // L, body, 0, unroll=8)

    for k in range(n_chunks):
      if k + 1 < n_chunks:
        descs[k + 1] = pltpu.async_copy(
            x_hbm.at[pl.ds(base + (k + 1) * CHUNK, CHUNK)],
            bufs[(k + 1) % 2], sems[(k + 1) % 2])
      descs[k].wait()
      count_body(bufs[k % 2])

    pltpu.sync_copy(hist, out_hbm.at[wid])

  return hist_kernel(pair_i)


def _tc_reduce(partials):
  nw = partials.shape[0]
  n_grid = PAD_N // (8 * 1024)

  def reduce_kernel(x_ref, o_ref):
    o_ref[...] = jnp.sum(x_ref[...], axis=0)

  out = pl.pallas_call(
      reduce_kernel,
      grid=(n_grid,),
      in_specs=[pl.BlockSpec((nw, 1, 8, 1024), lambda i: (0, i, 0, 0))],
      out_specs=pl.BlockSpec((1, 8, 1024), lambda i: (i, 0, 0)),
      out_shape=jax.ShapeDtypeStruct((n_grid, 8, 1024), jnp.int32),
  )(partials.reshape(nw, n_grid, 8, 1024))
  return out.reshape(PAD_N)


@jax.jit
def kernel(pair_i):
  pair_i = pair_i.astype(jnp.int32)
  partials = _sc_partial_hists(pair_i)
  return _tc_reduce(partials)[:N_ATOMS]


# speedup vs baseline: 1.7266x; 1.0918x over previous
import functools

import jax
import jax.numpy as jnp
from jax import lax
from jax.experimental import pallas as pl
from jax.experimental.pallas import tpu as pltpu
from jax.experimental.pallas import tpu_sc as plsc

N_ATOMS = 100000
N_PAIRS = 6400000
L = 16

PAD_N = 106496

CHUNK = 10000


def _sc_partial_hists(pair_i):
  mesh = plsc.VectorSubcoreMesh(core_axis_name="c", subcore_axis_name="s")
  nw = mesh.num_cores * mesh.num_subcores
  per_w = N_PAIRS // nw
  n_chunks = per_w // CHUNK

  @functools.partial(
      pl.kernel,
      out_type=jax.ShapeDtypeStruct((nw, PAD_N), jnp.int32),
      mesh=mesh,
      compiler_params=pltpu.CompilerParams(needs_layout_passes=False),
      scratch_types=[
          pltpu.VMEM((PAD_N,), jnp.int32),
          pltpu.VMEM((CHUNK + 32,), jnp.int32),
          pltpu.VMEM((CHUNK + 32,), jnp.int32),
          pltpu.SemaphoreType.DMA,
          pltpu.SemaphoreType.DMA,
      ],
  )
  def hist_kernel(x_hbm, out_hbm, hist, buf0, buf1, sem0, sem1):
    wid = lax.axis_index("s") * mesh.num_cores + lax.axis_index("c")
    base = wid * per_w

    bufs = (buf0, buf1)
    sems = (sem0, sem1)
    descs = [None] * n_chunks
    descs[0] = pltpu.async_copy(
        x_hbm.at[pl.ds(base, CHUNK)], buf0.at[pl.ds(16, CHUNK)], sem0)

    zv = jnp.zeros((L,), jnp.int32)

    def zero_body(j, c):
      hist[pl.ds(j * L, L)] = zv
      return c

    lax.fori_loop(0, PAD_N // L, zero_body, 0, unroll=8)

    iota = lax.iota(jnp.int32, L)
    lane0 = iota == 0
    lane15 = iota == L - 1

    def count_body(buf):
      def body(i, c):
        off = i * L + 16
        v = buf[pl.ds(off, L)]
        prv = buf[pl.ds(off - 1, L)]
        nxt = buf[pl.ds(off + 1, L)]
        m_start = (v != prv) | lane0
        m_end = (v != nxt) | lane15
        s = plsc.cummax(jnp.where(m_start, iota, 0))
        cnt = iota - s + 1
        plsc.addupdate_scatter(hist, [v], cnt, mask=m_end)
        return c
      lax.fori_loop(0, CHUNK // L, body, 0, unroll=8)

    for k in range(n_chunks):
      if k + 1 < n_chunks:
        descs[k + 1] = pltpu.async_copy(
            x_hbm.at[pl.ds(base + (k + 1) * CHUNK, CHUNK)],
            bufs[(k + 1) % 2].at[pl.ds(16, CHUNK)], sems[(k + 1) % 2])
      descs[k].wait()
      count_body(bufs[k % 2])

    pltpu.sync_copy(hist, out_hbm.at[wid])

  return hist_kernel(pair_i)


def _tc_reduce(partials):
  nw = partials.shape[0]
  n_grid = PAD_N // (8 * 1024)

  def reduce_kernel(x_ref, o_ref):
    o_ref[...] = jnp.sum(x_ref[...], axis=0)

  out = pl.pallas_call(
      reduce_kernel,
      grid=(n_grid,),
      in_specs=[pl.BlockSpec((nw, 1, 8, 1024), lambda i: (0, i, 0, 0))],
      out_specs=pl.BlockSpec((1, 8, 1024), lambda i: (i, 0, 0)),
      out_shape=jax.ShapeDtypeStruct((n_grid, 8, 1024), jnp.int32),
  )(partials.reshape(nw, n_grid, 8, 1024))
  return out.reshape(PAD_N)


@jax.jit
def kernel(pair_i):
  pair_i = pair_i.astype(jnp.int32)
  partials = _sc_partial_hists(pair_i)
  return _tc_reduce(partials)[:N_ATOMS]


# speedup vs baseline: 3.7951x; 2.1981x over previous
import functools

import jax
import jax.numpy as jnp
from jax import lax
from jax.experimental import pallas as pl
from jax.experimental.pallas import tpu as pltpu
from jax.experimental.pallas import tpu_sc as plsc

N_ATOMS = 100000
N_PAIRS = 6400000
L = 16

PAD_N = 106496

CHUNK = 10000


def _sc_partial_hists(pair_i):
  mesh = plsc.VectorSubcoreMesh(core_axis_name="c", subcore_axis_name="s")
  nw = mesh.num_cores * mesh.num_subcores
  per_w = N_PAIRS // nw
  n_chunks = per_w // CHUNK

  @functools.partial(
      pl.kernel,
      out_type=jax.ShapeDtypeStruct((nw, PAD_N), jnp.int32),
      mesh=mesh,
      compiler_params=pltpu.CompilerParams(needs_layout_passes=False),
      scratch_types=[
          pltpu.VMEM((PAD_N,), jnp.int32),
          pltpu.VMEM((CHUNK + 32,), jnp.int32),
          pltpu.VMEM((CHUNK + 32,), jnp.int32),
          pltpu.SemaphoreType.DMA,
          pltpu.SemaphoreType.DMA,
      ],
  )
  def hist_kernel(x_hbm, out_hbm, hist, buf0, buf1, sem0, sem1):
    wid = lax.axis_index("s") * mesh.num_cores + lax.axis_index("c")
    base = wid * per_w

    bufs = (buf0, buf1)
    sems = (sem0, sem1)
    descs = [None] * n_chunks
    descs[0] = pltpu.async_copy(
        x_hbm.at[pl.ds(base, CHUNK)], buf0.at[pl.ds(16, CHUNK)], sem0)

    zv = jnp.zeros((L,), jnp.int32)

    @plsc.parallel_loop(0, PAD_N // L, unroll=8)
    def _(j):
      hist[pl.ds(j * L, L)] = zv

    iota = lax.iota(jnp.int32, L)
    lane0 = iota == 0
    lane15 = iota == L - 1

    def count_body(buf):
      @plsc.parallel_loop(0, CHUNK // L, unroll=8)
      def _(i):
        off = i * L + 16
        v = buf[pl.ds(off, L)]
        prv = buf[pl.ds(off - 1, L)]
        nxt = buf[pl.ds(off + 1, L)]
        m_start = (v != prv) | lane0
        m_end = (v != nxt) | lane15
        s = plsc.cummax(jnp.where(m_start, iota, 0))
        cnt = iota - s + 1
        plsc.addupdate_scatter(hist, [v], cnt, mask=m_end)

    for k in range(n_chunks):
      if k + 1 < n_chunks:
        descs[k + 1] = pltpu.async_copy(
            x_hbm.at[pl.ds(base + (k + 1) * CHUNK, CHUNK)],
            bufs[(k + 1) % 2].at[pl.ds(16, CHUNK)], sems[(k + 1) % 2])
      descs[k].wait()
      count_body(bufs[k % 2])

    pltpu.sync_copy(hist, out_hbm.at[wid])

  return hist_kernel(pair_i)


def _tc_reduce(partials):
  nw = partials.shape[0]
  n_grid = PAD_N // (8 * 1024)

  def reduce_kernel(x_ref, o_ref):
    o_ref[...] = jnp.sum(x_ref[...], axis=0)

  out = pl.pallas_call(
      reduce_kernel,
      grid=(n_grid,),
      in_specs=[pl.BlockSpec((nw, 1, 8, 1024), lambda i: (0, i, 0, 0))],
      out_specs=pl.BlockSpec((1, 8, 1024), lambda i: (i, 0, 0)),
      out_shape=jax.ShapeDtypeStruct((n_grid, 8, 1024), jnp.int32),
  )(partials.reshape(nw, n_grid, 8, 1024))
  return out.reshape(PAD_N)


@jax.jit
def kernel(pair_i):
  pair_i = pair_i.astype(jnp.int32)
  partials = _sc_partial_hists(pair_i)
  return _tc_reduce(partials)[:N_ATOMS]


# speedup vs baseline: 4.5422x; 1.1969x over previous
import functools

import jax
import jax.numpy as jnp
from jax import lax
from jax.experimental import pallas as pl
from jax.experimental.pallas import tpu as pltpu
from jax.experimental.pallas import tpu_sc as plsc

N_ATOMS = 100000
N_PAIRS = 6400000
L = 16

PAD_N = 106496

CHUNK = 10000


def _sc_partial_hists(pair_i):
  mesh = plsc.VectorSubcoreMesh(core_axis_name="c", subcore_axis_name="s")
  nw = mesh.num_cores * mesh.num_subcores
  per_w = N_PAIRS // nw
  n_chunks = per_w // CHUNK

  @functools.partial(
      pl.kernel,
      out_type=jax.ShapeDtypeStruct((nw, PAD_N), jnp.int32),
      mesh=mesh,
      compiler_params=pltpu.CompilerParams(needs_layout_passes=False),
      scratch_types=[
          pltpu.VMEM((PAD_N,), jnp.int32),
          pltpu.VMEM((CHUNK + 32,), jnp.int32),
          pltpu.VMEM((CHUNK + 32,), jnp.int32),
          pltpu.SemaphoreType.DMA,
          pltpu.SemaphoreType.DMA,
      ],
  )
  def hist_kernel(x_hbm, out_hbm, hist, buf0, buf1, sem0, sem1):
    wid = lax.axis_index("s") * mesh.num_cores + lax.axis_index("c")
    base = wid * per_w

    bufs = (buf0, buf1)
    sems = (sem0, sem1)
    descs = [None] * n_chunks
    descs[0] = pltpu.async_copy(
        x_hbm.at[pl.ds(base, CHUNK)], buf0.at[pl.ds(16, CHUNK)], sem0)

    zv = jnp.zeros((L,), jnp.int32)

    @plsc.parallel_loop(0, PAD_N // L, unroll=8)
    def _(j):
      hist[pl.ds(j * L, L)] = zv

    iota = lax.iota(jnp.int32, L)
    lane0 = iota == 0
    lane15 = iota == L - 1

    def count_body(buf):
      @plsc.parallel_loop(0, CHUNK // L, unroll=8)
      def _(i):
        off = i * L + 16
        v = buf[pl.ds(off, L)]
        prv = buf[pl.ds(off - 1, L)]
        nxt = buf[pl.ds(off + 1, L)]
        m_start = (v != prv) | lane0
        m_end = (v != nxt) | lane15
        s = plsc.cummax(jnp.where(m_start, iota, 0))
        cnt = iota - s + 1
        plsc.addupdate_scatter(hist, [v], cnt, mask=m_end)

    for k in range(n_chunks):
      if k + 1 < n_chunks:
        descs[k + 1] = pltpu.async_copy(
            x_hbm.at[pl.ds(base + (k + 1) * CHUNK, CHUNK)],
            bufs[(k + 1) % 2].at[pl.ds(16, CHUNK)], sems[(k + 1) % 2])
      descs[k].wait()
      count_body(bufs[k % 2])

    pltpu.sync_copy(hist, out_hbm.at[wid])

  return hist_kernel(pair_i)


def _tc_reduce(partials):
  nw = partials.shape[0]
  blk = 8192
  n_grid = PAD_N // blk

  def reduce_kernel(x_ref, o_ref):
    o_ref[...] = jnp.sum(x_ref[...], axis=0)

  return pl.pallas_call(
      reduce_kernel,
      grid=(n_grid,),
      in_specs=[pl.BlockSpec((nw, blk), lambda i: (0, i))],
      out_specs=pl.BlockSpec((blk,), lambda i: (i,)),
      out_shape=jax.ShapeDtypeStruct((PAD_N,), jnp.int32),
  )(partials)


@jax.jit
def kernel(pair_i):
  pair_i = pair_i.astype(jnp.int32)
  partials = _sc_partial_hists(pair_i)
  return _tc_reduce(partials)[:N_ATOMS]
